# Initial kernel scaffold; baseline (speedup 1.0000x reference)
import jax
import jax.numpy as jnp
from jax.experimental import pallas as pl


def kernel(x, edge_index, W1, b1, W2, b2, W3, b3, W4, b4):
    raise NotImplementedError("write your pallas kernel here")


# speedup vs baseline: 17.7116x; 17.7116x over previous
import dataclasses
import functools

import jax
import jax.numpy as jnp
from jax import lax
from jax.experimental import pallas as pl
from jax.experimental.pallas import tpu as pltpu
from jax.experimental.pallas import tpu_sc as plsc

N = 10000
NP = 10240
E = 320000
NC = 2
NS = 16
NW = NC * NS
EPW = E // NW
CH = 125
NCH = EPW // CH
RPT = NP // NS

_MESH = plsc.VectorSubcoreMesh(core_axis_name="c", subcore_axis_name="s")

_SC_CP = pltpu.CompilerParams()
if "needs_layout_passes" in pltpu.CompilerParams.__dataclass_fields__:
    _SC_CP = dataclasses.replace(_SC_CP, needs_layout_passes=False)


@functools.partial(
    pl.kernel,
    out_type=jax.ShapeDtypeStruct((NW, NP), jnp.float32),
    mesh=_MESH,
    compiler_params=_SC_CP,
    scratch_types=[
        pltpu.VMEM((EPW,), jnp.int32),
        pltpu.VMEM((NP,), jnp.float32),
    ],
)
def _deg_kernel(dst_hbm, out_hbm, dstv, hist):
    c = lax.axis_index("c")
    s = lax.axis_index("s")
    wid = c * NS + s
    pltpu.sync_copy(dst_hbm.at[wid], dstv)

    @pl.loop(0, NP, step=16)
    def _(i):
        hist[pl.ds(i, 16)] = jnp.zeros((16,), jnp.float32)

    ones = jnp.full((16,), 1.0, jnp.float32)

    @pl.loop(0, EPW, step=16)
    def _(i):
        idx = dstv[pl.ds(i, 16)]
        plsc.addupdate_scatter(hist, [idx], ones)

    pltpu.sync_copy(hist, out_hbm.at[wid])


def _make_agg(D):
    @functools.partial(
        pl.kernel,
        out_type=jax.ShapeDtypeStruct((NC, NP, D), jnp.float32),
        mesh=_MESH,
        scratch_types=[
            pltpu.VMEM((NCH, CH), jnp.int32),
            pltpu.VMEM((NCH, CH), jnp.int32),
            pltpu.VMEM((CH, D), jnp.float32),
            pltpu.VMEM_SHARED((NP, D), jnp.float32),
            pltpu.SemaphoreType.DMA,
        ],
    )
    def agg(h_hbm, src_hbm, dst_hbm, out_hbm, srcv, dstv, rows, acc, sem):
        c = lax.axis_index("c")
        s = lax.axis_index("s")
        wid = c * NS + s
        rs = s * RPT
        pltpu.sync_copy(h_hbm.at[pl.ds(rs, RPT)], acc.at[pl.ds(rs, RPT)])
        pltpu.sync_copy(src_hbm.at[wid], srcv)
        pltpu.sync_copy(dst_hbm.at[wid], dstv)
        plsc.subcore_barrier()

        @pl.loop(0, NCH)
        def _(k):
            pltpu.async_copy(h_hbm.at[srcv.at[k]], rows, sem).wait()
            pltpu.sync_copy(rows, acc.at[dstv.at[k]], add=True)

        plsc.subcore_barrier()
        pltpu.sync_copy(acc.at[pl.ds(rs, RPT)], out_hbm.at[c, pl.ds(rs, RPT)])

    return agg


_agg128 = _make_agg(128)


_R = 2048


def _dinv_of(hist_blk):
    return lax.rsqrt(jnp.sum(hist_blk, axis=0) + 1.0)


def _tc_first(hist, xp, W1):
    def body(hist_ref, x_ref, w_ref, out_ref):
        dinv = _dinv_of(hist_ref[...])
        h = jnp.dot(x_ref[...], w_ref[...], preferred_element_type=jnp.float32)
        out_ref[...] = h * dinv[:, None]

    return pl.pallas_call(
        body,
        grid=(NP // _R,),
        in_specs=[
            pl.BlockSpec((NW, _R), lambda i: (0, i)),
            pl.BlockSpec((_R, 128), lambda i: (i, 0)),
            pl.BlockSpec((128, 128), lambda i: (0, 0)),
        ],
        out_specs=pl.BlockSpec((_R, 128), lambda i: (i, 0)),
        out_shape=jax.ShapeDtypeStruct((NP, 128), jnp.float32),
    )(hist, xp, W1)


def _tc_mid(hist, p, hp, b_prev, W=None):
    def body(hist_ref, p_ref, h_ref, b_ref, w_ref, out_ref):
        dinv = _dinv_of(hist_ref[...])
        sgg = (p_ref[0] + p_ref[1] - h_ref[...]) * dinv[:, None] + b_ref[...]
        a = jnp.maximum(sgg, 0.0)
        if w_ref is not None:
            a = jnp.dot(a, w_ref[...], preferred_element_type=jnp.float32)
        out_ref[...] = a * dinv[:, None]

    in_specs = [
        pl.BlockSpec((NW, _R), lambda i: (0, i)),
        pl.BlockSpec((NC, _R, 128), lambda i: (0, i, 0)),
        pl.BlockSpec((_R, 128), lambda i: (i, 0)),
        pl.BlockSpec((1, 128), lambda i: (0, 0)),
    ]
    args = [hist, p, hp, b_prev]
    if W is not None:
        in_specs.append(pl.BlockSpec((128, 128), lambda i: (0, 0)))
        args.append(W)
        fn = body
    else:
        fn = lambda h_, p_, hp_, b_, o_: body(h_, p_, hp_, b_, None, o_)

    return pl.pallas_call(
        fn,
        grid=(NP // _R,),
        in_specs=in_specs,
        out_specs=pl.BlockSpec((_R, 128), lambda i: (i, 0)),
        out_shape=jax.ShapeDtypeStruct((NP, 128), jnp.float32),
    )(*args)


def _tc_final(hist, p, hp, W4, b4):
    def body(hist_ref, p_ref, h_ref, w_ref, b_ref, out_ref):
        dinv = _dinv_of(hist_ref[...])
        agg = (p_ref[0] + p_ref[1] - h_ref[...]) * dinv[:, None]
        sgg = jnp.dot(agg, w_ref[...], preferred_element_type=jnp.float32)
        sgg = sgg + b_ref[...]
        m = jnp.max(sgg, axis=1, keepdims=True)
        sh = sgg - m
        lse = jnp.log(jnp.sum(jnp.exp(sh), axis=1, keepdims=True))
        out_ref[...] = sh - lse

    return pl.pallas_call(
        body,
        grid=(NP // _R,),
        in_specs=[
            pl.BlockSpec((NW, _R), lambda i: (0, i)),
            pl.BlockSpec((NC, _R, 128), lambda i: (0, i, 0)),
            pl.BlockSpec((_R, 128), lambda i: (i, 0)),
            pl.BlockSpec((128, 16), lambda i: (0, 0)),
            pl.BlockSpec((1, 16), lambda i: (0, 0)),
        ],
        out_specs=pl.BlockSpec((_R, 16), lambda i: (i, 0)),
        out_shape=jax.ShapeDtypeStruct((NP, 16), jnp.float32),
    )(hist, p, hp, W4, b4)


def kernel(x, edge_index, W1, b1, W2, b2, W3, b3, W4, b4):
    ei = edge_index.astype(jnp.int32)
    src_r = ei[0].reshape(NW, NCH, CH)
    dst_r = ei[1].reshape(NW, NCH, CH)
    dst_flat = ei[1].reshape(NW, EPW)

    xp = jnp.pad(x, ((0, NP - N), (0, 0)))
    b1r, b2r, b3r = b1.reshape(1, 128), b2.reshape(1, 128), b3.reshape(1, 128)
    b4r = b4.reshape(1, 16)

    hist = _deg_kernel(dst_flat)

    hp1 = _tc_first(hist, xp, W1)
    p1 = _agg128(hp1, src_r, dst_r)
    hp2 = _tc_mid(hist, p1, hp1, b1r, W2)
    p2 = _agg128(hp2, src_r, dst_r)
    hp3 = _tc_mid(hist, p2, hp2, b2r, W3)
    p3 = _agg128(hp3, src_r, dst_r)
    g3 = _tc_mid(hist, p3, hp3, b3r, None)
    p4 = _agg128(g3, src_r, dst_r)
    out = _tc_final(hist, p4, g3, W4, b4r)
    return out[:N]


# speedup vs baseline: 20.0311x; 1.1310x over previous
import dataclasses
import functools

import jax
import jax.numpy as jnp
from jax import lax
from jax.experimental import pallas as pl
from jax.experimental.pallas import tpu as pltpu
from jax.experimental.pallas import tpu_sc as plsc

N = 10000
NP = 10240
E = 320000
NC = 2
NS = 16
NW = NC * NS
EPW = E // NW
HALF = NP // NC
ACCR = HALF + 8
RPT = HALF // NS

ES = E // NS
ESR = 250
ESC = 80
CH = 128
MAXCH = 160
CAP = MAXCH * CH

_MESH = plsc.VectorSubcoreMesh(core_axis_name="c", subcore_axis_name="s")

_SC_CP = pltpu.CompilerParams()
if "needs_layout_passes" in pltpu.CompilerParams.__dataclass_fields__:
    _SC_CP = dataclasses.replace(_SC_CP, needs_layout_passes=False)


@functools.partial(
    pl.kernel,
    out_type=jax.ShapeDtypeStruct((NW, NP), jnp.float32),
    mesh=_MESH,
    compiler_params=_SC_CP,
    scratch_types=[
        pltpu.VMEM((EPW,), jnp.int32),
        pltpu.VMEM((NP,), jnp.float32),
    ],
)
def _deg_kernel(dst_hbm, out_hbm, dstv, hist):
    c = lax.axis_index("c")
    s = lax.axis_index("s")
    wid = c * NS + s
    pltpu.sync_copy(dst_hbm.at[wid], dstv)

    @pl.loop(0, NP, step=16)
    def _(i):
        hist[pl.ds(i, 16)] = jnp.zeros((16,), jnp.float32)

    ones = jnp.full((16,), 1.0, jnp.float32)

    @pl.loop(0, EPW, step=16)
    def _(i):
        idx = dstv[pl.ds(i, 16)]
        plsc.addupdate_scatter(hist, [idx], ones)

    pltpu.sync_copy(hist, out_hbm.at[wid])


@functools.partial(
    pl.kernel,
    out_type=(
        jax.ShapeDtypeStruct((NC, NS, 2, CAP), jnp.int32),
        jax.ShapeDtypeStruct((NC, NS, 16), jnp.int32),
    ),
    mesh=_MESH,
    compiler_params=_SC_CP,
    scratch_types=[
        pltpu.VMEM((2, ESR, ESC), jnp.int32),
        pltpu.VMEM((CAP,), jnp.int32),
        pltpu.VMEM((CAP,), jnp.int32),
        pltpu.VMEM((16,), jnp.int32),
        pltpu.SMEM((1,), jnp.int32),
    ],
)
def _comp_kernel(sd_hbm, comp_hbm, np_hbm, raw, comp_s, comp_d, npv, offs):
    c = lax.axis_index("c")
    s = lax.axis_index("s")
    lo = c * HALF
    pltpu.sync_copy(sd_hbm.at[s], raw)
    offs[0] = 0
    lane = lax.iota(jnp.int32, 16)

    @pl.loop(0, ESR)
    def _(r):
        @pl.loop(0, ESC, step=16)
        def _(j):
            sv = raw.at[0].at[r][pl.ds(j, 16)]
            dv = raw.at[1].at[r][pl.ds(j, 16)]
            dl = dv - lo
            ok = (dl >= 0) & (dl < HALF)
            oki = ok.astype(jnp.int32)
            cum = lax.cumsum(oki)
            pos = offs[0] + cum - oki
            plsc.store_scatter(comp_s, [pos], sv, mask=ok)
            plsc.store_scatter(comp_d, [pos], dl, mask=ok)
            offs[0] = offs[0] + jnp.sum(oki)

    cnt = offs[0]

    @pl.loop(0, 16)
    def _(j):
        pos = cnt + j * 16 + lane
        plsc.store_scatter(comp_s, [pos], (j * 16 + lane) * 8)
        plsc.store_scatter(comp_d, [pos], HALF + (lane & 7))

    npairs = (cnt + 2 * CH - 1) // (2 * CH)
    npv[...] = jnp.zeros((16,), jnp.int32) + npairs
    pltpu.sync_copy(npv, np_hbm.at[c, s])
    pltpu.sync_copy(comp_s, comp_hbm.at[c, s, 0])
    pltpu.sync_copy(comp_d, comp_hbm.at[c, s, 1])


@functools.partial(
    pl.kernel,
    out_type=jax.ShapeDtypeStruct((NP, 128), jnp.float32),
    mesh=_MESH,
    compiler_params=_SC_CP,
    scratch_types=[
        pltpu.VMEM((2, MAXCH, CH), jnp.int32),
        pltpu.VMEM((2, CH, 128), jnp.float32),
        pltpu.VMEM_SHARED((ACCR, 128), jnp.float32),
        pltpu.VMEM((16,), jnp.int32),
        pltpu.SemaphoreType.DMA,
        pltpu.SemaphoreType.DMA,
    ],
)
def _agg_kernel(h_hbm, comp_hbm, np_hbm, out_hbm, compv, rows, acc, npv,
                sem_a, sem_b):
    c = lax.axis_index("c")
    s = lax.axis_index("s")
    lo = c * HALF
    rs = s * RPT
    pltpu.sync_copy(h_hbm.at[pl.ds(lo + rs, RPT)], acc.at[pl.ds(rs, RPT)])
    pltpu.sync_copy(comp_hbm.at[c, s], compv)
    pltpu.sync_copy(np_hbm.at[c, s], npv)
    srcv = compv.at[0]
    dstv = compv.at[1]
    npairs = jnp.minimum(jnp.max(npv[...]), MAXCH // 2)
    plsc.subcore_barrier()

    @pl.loop(0, npairs)
    def _(j):
        k = j * 2
        ga = pltpu.async_copy(h_hbm.at[srcv.at[k]], rows.at[0], sem_a)
        gb = pltpu.async_copy(h_hbm.at[srcv.at[k + 1]], rows.at[1], sem_b)
        ga.wait()
        pltpu.sync_copy(rows.at[0], acc.at[dstv.at[k]], add=True)
        gb.wait()
        pltpu.sync_copy(rows.at[1], acc.at[dstv.at[k + 1]], add=True)

    plsc.subcore_barrier()
    pltpu.sync_copy(acc.at[pl.ds(rs, RPT)], out_hbm.at[pl.ds(lo + rs, RPT)])


_R = 2048


def _dinv_of(hist_blk):
    return lax.rsqrt(jnp.sum(hist_blk, axis=0) + 1.0)


def _tc_first(hist, xp, W1):
    def body(hist_ref, x_ref, w_ref, out_ref):
        dinv = _dinv_of(hist_ref[...])
        h = jnp.dot(x_ref[...], w_ref[...], preferred_element_type=jnp.float32,
                    precision=lax.Precision.HIGHEST)
        out_ref[...] = h * dinv[:, None]

    return pl.pallas_call(
        body,
        grid=(NP // _R,),
        in_specs=[
            pl.BlockSpec((NW, _R), lambda i: (0, i)),
            pl.BlockSpec((_R, 128), lambda i: (i, 0)),
            pl.BlockSpec((128, 128), lambda i: (0, 0)),
        ],
        out_specs=pl.BlockSpec((_R, 128), lambda i: (i, 0)),
        out_shape=jax.ShapeDtypeStruct((NP, 128), jnp.float32),
    )(hist, xp, W1)


def _tc_mid(hist, p, hp, b_prev, W=None):
    def body(hist_ref, p_ref, h_ref, b_ref, w_ref, out_ref):
        dinv = _dinv_of(hist_ref[...])
        sgg = (p_ref[...] - h_ref[...]) * dinv[:, None] + b_ref[...]
        a = jnp.maximum(sgg, 0.0)
        if w_ref is not None:
            a = jnp.dot(a, w_ref[...], preferred_element_type=jnp.float32,
                        precision=lax.Precision.HIGHEST)
        out_ref[...] = a * dinv[:, None]

    in_specs = [
        pl.BlockSpec((NW, _R), lambda i: (0, i)),
        pl.BlockSpec((_R, 128), lambda i: (i, 0)),
        pl.BlockSpec((_R, 128), lambda i: (i, 0)),
        pl.BlockSpec((1, 128), lambda i: (0, 0)),
    ]
    args = [hist, p, hp, b_prev]
    if W is not None:
        in_specs.append(pl.BlockSpec((128, 128), lambda i: (0, 0)))
        args.append(W)
        fn = body
    else:
        fn = lambda h_, p_, hp_, b_, o_: body(h_, p_, hp_, b_, None, o_)

    return pl.pallas_call(
        fn,
        grid=(NP // _R,),
        in_specs=in_specs,
        out_specs=pl.BlockSpec((_R, 128), lambda i: (i, 0)),
        out_shape=jax.ShapeDtypeStruct((NP, 128), jnp.float32),
    )(*args)


def _tc_final(hist, p, hp, W4, b4):
    def body(hist_ref, p_ref, h_ref, w_ref, b_ref, out_ref):
        dinv = _dinv_of(hist_ref[...])
        agg = (p_ref[...] - h_ref[...]) * dinv[:, None]
        sgg = jnp.dot(agg, w_ref[...], preferred_element_type=jnp.float32,
                      precision=lax.Precision.HIGHEST)
        sgg = sgg + b_ref[...]
        m = jnp.max(sgg, axis=1, keepdims=True)
        sh = sgg - m
        lse = jnp.log(jnp.sum(jnp.exp(sh), axis=1, keepdims=True))
        out_ref[...] = sh - lse

    return pl.pallas_call(
        body,
        grid=(NP // _R,),
        in_specs=[
            pl.BlockSpec((NW, _R), lambda i: (0, i)),
            pl.BlockSpec((_R, 128), lambda i: (i, 0)),
            pl.BlockSpec((_R, 128), lambda i: (i, 0)),
            pl.BlockSpec((128, 16), lambda i: (0, 0)),
            pl.BlockSpec((1, 16), lambda i: (0, 0)),
        ],
        out_specs=pl.BlockSpec((_R, 16), lambda i: (i, 0)),
        out_shape=jax.ShapeDtypeStruct((NP, 16), jnp.float32),
    )(hist, p, hp, W4, b4)


def kernel(x, edge_index, W1, b1, W2, b2, W3, b3, W4, b4):
    ei = edge_index.astype(jnp.int32)
    sd = ei.reshape(2, NS, ESR, ESC).transpose(1, 0, 2, 3)
    dst_flat = ei[1].reshape(NW, EPW)

    xp = jnp.pad(x, ((0, NP - N), (0, 0)))
    b1r, b2r, b3r = b1.reshape(1, 128), b2.reshape(1, 128), b3.reshape(1, 128)

    hist = _deg_kernel(dst_flat)
    comp, npair = _comp_kernel(sd)
    comp = comp.reshape(NC, NS, 2, MAXCH, CH)

    hp1 = _tc_first(hist, xp, W1)
    p1 = _agg_kernel(hp1, comp, npair)
    hp2 = _tc_mid(hist, p1, hp1, b1r, W2)
    p2 = _agg_kernel(hp2, comp, npair)
    hp3 = _tc_mid(hist, p2, hp2, b2r, W3)
    p3 = _agg_kernel(hp3, comp, npair)
    g3 = _tc_mid(hist, p3, hp3, b3r, None)
    p4 = _agg_kernel(g3, comp, npair)
    out = _tc_final(hist, p4, g3, W4, b4.reshape(1, 16))
    return out[:N]


# speedup vs baseline: 26.8620x; 1.3410x over previous
import dataclasses
import functools

import jax
import jax.numpy as jnp
from jax import lax
from jax.experimental import pallas as pl
from jax.experimental.pallas import tpu as pltpu
from jax.experimental.pallas import tpu_sc as plsc

N = 10000
NP = 10240
E = 320000
NC = 2
NS = 16
NW = NC * NS
EPW = E // NW
HALF = NP // NC
ACCR = HALF + 8
RPT = HALF // NS

ES = E // NS
ESR = 250
ESC = 80
CH = 128
MAXCH = 160
CAP = MAXCH * CH

_MESH = plsc.VectorSubcoreMesh(core_axis_name="c", subcore_axis_name="s")

_SC_CP = pltpu.CompilerParams()
if "needs_layout_passes" in pltpu.CompilerParams.__dataclass_fields__:
    _SC_CP = dataclasses.replace(_SC_CP, needs_layout_passes=False)


@functools.partial(
    pl.kernel,
    out_type=jax.ShapeDtypeStruct((NW, NP), jnp.float32),
    mesh=_MESH,
    compiler_params=_SC_CP,
    scratch_types=[
        pltpu.VMEM((EPW,), jnp.int32),
        pltpu.VMEM((NP,), jnp.float32),
    ],
)
def _deg_kernel(dst_hbm, out_hbm, dstv, hist):
    c = lax.axis_index("c")
    s = lax.axis_index("s")
    wid = c * NS + s
    pltpu.sync_copy(dst_hbm.at[wid], dstv)

    @pl.loop(0, NP, step=16)
    def _(i):
        hist[pl.ds(i, 16)] = jnp.zeros((16,), jnp.float32)

    ones = jnp.full((16,), 1.0, jnp.float32)

    @pl.loop(0, EPW, step=16)
    def _(i):
        idx = dstv[pl.ds(i, 16)]
        plsc.addupdate_scatter(hist, [idx], ones)

    pltpu.sync_copy(hist, out_hbm.at[wid])


@functools.partial(
    pl.kernel,
    out_type=(
        jax.ShapeDtypeStruct((NC, NS, 2, CAP), jnp.int32),
        jax.ShapeDtypeStruct((NC, NS, 16), jnp.int32),
    ),
    mesh=_MESH,
    compiler_params=_SC_CP,
    scratch_types=[
        pltpu.VMEM((2, ESR, ESC), jnp.int32),
        pltpu.VMEM((CAP,), jnp.int32),
        pltpu.VMEM((CAP,), jnp.int32),
        pltpu.VMEM((16,), jnp.int32),
        pltpu.SMEM((1,), jnp.int32),
    ],
)
def _comp_kernel(sd_hbm, comp_hbm, np_hbm, raw, comp_s, comp_d, npv, offs):
    c = lax.axis_index("c")
    s = lax.axis_index("s")
    lo = c * HALF
    pltpu.sync_copy(sd_hbm.at[s], raw)
    offs[0] = 0
    lane = lax.iota(jnp.int32, 16)

    @pl.loop(0, ESR)
    def _(r):
        @pl.loop(0, ESC, step=16)
        def _(j):
            sv = raw.at[0].at[r][pl.ds(j, 16)]
            dv = raw.at[1].at[r][pl.ds(j, 16)]
            dl = dv - lo
            ok = (dl >= 0) & (dl < HALF)
            oki = ok.astype(jnp.int32)
            cum = lax.cumsum(oki)
            pos = offs[0] + cum - oki
            plsc.store_scatter(comp_s, [pos], sv, mask=ok)
            plsc.store_scatter(comp_d, [pos], dl, mask=ok)
            offs[0] = offs[0] + jnp.sum(oki)

    cnt = offs[0]

    @pl.loop(0, 16)
    def _(j):
        pos = cnt + j * 16 + lane
        plsc.store_scatter(comp_s, [pos], (j * 16 + lane) * 8)
        plsc.store_scatter(comp_d, [pos], HALF + (lane & 7))

    npairs = (cnt + 2 * CH - 1) // (2 * CH)
    npv[...] = jnp.zeros((16,), jnp.int32) + npairs
    pltpu.sync_copy(npv, np_hbm.at[c, s])
    pltpu.sync_copy(comp_s, comp_hbm.at[c, s, 0])
    pltpu.sync_copy(comp_d, comp_hbm.at[c, s, 1])


@functools.partial(
    pl.kernel,
    out_type=jax.ShapeDtypeStruct((NP, 128), jnp.float32),
    mesh=_MESH,
    compiler_params=_SC_CP,
    scratch_types=[
        pltpu.VMEM((2, MAXCH, CH), jnp.int32),
        pltpu.VMEM((2, CH, 128), jnp.float32),
        pltpu.VMEM_SHARED((ACCR, 128), jnp.float32),
        pltpu.VMEM((16,), jnp.int32),
        pltpu.SemaphoreType.DMA,
        pltpu.SemaphoreType.DMA,
    ],
)
def _agg_kernel(h_hbm, comp_hbm, np_hbm, out_hbm, compv, rows, acc, npv,
                sem_a, sem_b):
    c = lax.axis_index("c")
    s = lax.axis_index("s")
    lo = c * HALF
    rs = s * RPT
    pltpu.sync_copy(h_hbm.at[pl.ds(lo + rs, RPT)], acc.at[pl.ds(rs, RPT)])
    pltpu.sync_copy(comp_hbm.at[c, s], compv)
    pltpu.sync_copy(np_hbm.at[c, s], npv)
    srcv = compv.at[0]
    dstv = compv.at[1]
    npairs = jnp.minimum(jnp.max(npv[...]), MAXCH // 2)
    nch = npairs * 2

    @pl.when(npairs > 0)
    def _():
        pltpu.async_copy(h_hbm.at[srcv.at[0]], rows.at[0], sem_a)
        pltpu.async_copy(h_hbm.at[srcv.at[1]], rows.at[1], sem_b)

    plsc.subcore_barrier()

    @pl.loop(0, npairs)
    def _(j):
        k = j * 2
        pltpu.make_async_copy(h_hbm.at[srcv.at[k]], rows.at[0], sem_a).wait()
        pltpu.sync_copy(rows.at[0], acc.at[dstv.at[k]], add=True)

        @pl.when(k + 2 < nch)
        def _():
            pltpu.async_copy(h_hbm.at[srcv.at[k + 2]], rows.at[0], sem_a)

        pltpu.make_async_copy(
            h_hbm.at[srcv.at[k + 1]], rows.at[1], sem_b).wait()
        pltpu.sync_copy(rows.at[1], acc.at[dstv.at[k + 1]], add=True)

        @pl.when(k + 3 < nch)
        def _():
            pltpu.async_copy(h_hbm.at[srcv.at[k + 3]], rows.at[1], sem_b)

    plsc.subcore_barrier()
    pltpu.sync_copy(acc.at[pl.ds(rs, RPT)], out_hbm.at[pl.ds(lo + rs, RPT)])


_R = 2048


def _dinv_of(hist_blk):
    return lax.rsqrt(jnp.sum(hist_blk, axis=0) + 1.0)


def _tc_first(hist, xp, W1):
    def body(hist_ref, x_ref, w_ref, out_ref):
        dinv = _dinv_of(hist_ref[...])
        h = jnp.dot(x_ref[...], w_ref[...], preferred_element_type=jnp.float32,
                    precision=lax.Precision.HIGHEST)
        out_ref[...] = h * dinv[:, None]

    return pl.pallas_call(
        body,
        grid=(NP // _R,),
        in_specs=[
            pl.BlockSpec((NW, _R), lambda i: (0, i)),
            pl.BlockSpec((_R, 128), lambda i: (i, 0)),
            pl.BlockSpec((128, 128), lambda i: (0, 0)),
        ],
        out_specs=pl.BlockSpec((_R, 128), lambda i: (i, 0)),
        out_shape=jax.ShapeDtypeStruct((NP, 128), jnp.float32),
    )(hist, xp, W1)


def _tc_mid(hist, p, hp, b_prev, W=None):
    def body(hist_ref, p_ref, h_ref, b_ref, w_ref, out_ref):
        dinv = _dinv_of(hist_ref[...])
        sgg = (p_ref[...] - h_ref[...]) * dinv[:, None] + b_ref[...]
        a = jnp.maximum(sgg, 0.0)
        if w_ref is not None:
            a = jnp.dot(a, w_ref[...], preferred_element_type=jnp.float32,
                        precision=lax.Precision.HIGHEST)
        out_ref[...] = a * dinv[:, None]

    in_specs = [
        pl.BlockSpec((NW, _R), lambda i: (0, i)),
        pl.BlockSpec((_R, 128), lambda i: (i, 0)),
        pl.BlockSpec((_R, 128), lambda i: (i, 0)),
        pl.BlockSpec((1, 128), lambda i: (0, 0)),
    ]
    args = [hist, p, hp, b_prev]
    if W is not None:
        in_specs.append(pl.BlockSpec((128, 128), lambda i: (0, 0)))
        args.append(W)
        fn = body
    else:
        fn = lambda h_, p_, hp_, b_, o_: body(h_, p_, hp_, b_, None, o_)

    return pl.pallas_call(
        fn,
        grid=(NP // _R,),
        in_specs=in_specs,
        out_specs=pl.BlockSpec((_R, 128), lambda i: (i, 0)),
        out_shape=jax.ShapeDtypeStruct((NP, 128), jnp.float32),
    )(*args)


def _tc_final(hist, p, hp, W4, b4):
    def body(hist_ref, p_ref, h_ref, w_ref, b_ref, out_ref):
        dinv = _dinv_of(hist_ref[...])
        agg = (p_ref[...] - h_ref[...]) * dinv[:, None]
        sgg = jnp.dot(agg, w_ref[...], preferred_element_type=jnp.float32,
                      precision=lax.Precision.HIGHEST)
        sgg = sgg + b_ref[...]
        m = jnp.max(sgg, axis=1, keepdims=True)
        sh = sgg - m
        lse = jnp.log(jnp.sum(jnp.exp(sh), axis=1, keepdims=True))
        out_ref[...] = sh - lse

    return pl.pallas_call(
        body,
        grid=(NP // _R,),
        in_specs=[
            pl.BlockSpec((NW, _R), lambda i: (0, i)),
            pl.BlockSpec((_R, 128), lambda i: (i, 0)),
            pl.BlockSpec((_R, 128), lambda i: (i, 0)),
            pl.BlockSpec((128, 16), lambda i: (0, 0)),
            pl.BlockSpec((1, 16), lambda i: (0, 0)),
        ],
        out_specs=pl.BlockSpec((_R, 16), lambda i: (i, 0)),
        out_shape=jax.ShapeDtypeStruct((NP, 16), jnp.float32),
    )(hist, p, hp, W4, b4)


def kernel(x, edge_index, W1, b1, W2, b2, W3, b3, W4, b4):
    ei = edge_index.astype(jnp.int32)
    sd = ei.reshape(2, NS, ESR, ESC).transpose(1, 0, 2, 3)
    dst_flat = ei[1].reshape(NW, EPW)

    xp = jnp.pad(x, ((0, NP - N), (0, 0)))
    b1r, b2r, b3r = b1.reshape(1, 128), b2.reshape(1, 128), b3.reshape(1, 128)

    hist = _deg_kernel(dst_flat)
    comp, npair = _comp_kernel(sd)
    comp = comp.reshape(NC, NS, 2, MAXCH, CH)

    hp1 = _tc_first(hist, xp, W1)
    p1 = _agg_kernel(hp1, comp, npair)
    hp2 = _tc_mid(hist, p1, hp1, b1r, W2)
    p2 = _agg_kernel(hp2, comp, npair)
    hp3 = _tc_mid(hist, p2, hp2, b2r, W3)
    p3 = _agg_kernel(hp3, comp, npair)
    g3 = _tc_mid(hist, p3, hp3, b3r, None)
    p4 = _agg_kernel(g3, comp, npair)
    out = _tc_final(hist, p4, g3, W4, b4.reshape(1, 16))
    return out[:N]
